# baseline (speedup 1.0000x reference)
import functools

import jax
import jax.numpy as jnp
from jax import lax
from jax.experimental import pallas as pl
from jax.experimental.pallas import tpu as pltpu
from jax.experimental.pallas import tpu_sc as plsc

B = 100
T = 100
D = 128
H = 128
N = B * T

NC = 2
NS = 16
NTILES = NC * NS
CHUNK = 128
CPT = 80
SPC = 40
EPT = CPT * CHUNK
E_PAD = NTILES * EPT
NPAD = 10240
SLICE = NPAD // NS

_MESH = plsc.VectorSubcoreMesh(core_axis_name="c", subcore_axis_name="s")


def _gru_body(xs_ref, wih_ref, whh_ref, bih_ref, bhh_ref,
              out_ref, h_ref, gi_ref):
    gi_ref[...] = (jnp.dot(
        xs_ref[...].reshape(T * B, D), wih_ref[...],
        preferred_element_type=jnp.float32) + bih_ref[...]).reshape(
            T, B, 3 * H)
    h_ref[...] = jnp.zeros((B, H), dtype=jnp.float32)

    def step(t, carry):
        gi = gi_ref[t]
        gh = jnp.dot(h_ref[...], whh_ref[...], preferred_element_type=jnp.float32) + bhh_ref[...]
        r = jax.nn.sigmoid(gi[:, 0:H] + gh[:, 0:H])
        z = jax.nn.sigmoid(gi[:, H:2 * H] + gh[:, H:2 * H])
        n = jnp.tanh(gi[:, 2 * H:3 * H] + r * gh[:, 2 * H:3 * H])
        h_new = (1.0 - z) * n + z * h_ref[...]
        h_ref[...] = h_new
        out_ref[t] = h_new
        return carry

    lax.fori_loop(0, T, step, 0)


def _gru(xs, w_ih, w_hh, b_ih, b_hh):
    return pl.pallas_call(
        _gru_body,
        out_shape=jax.ShapeDtypeStruct((T, B, H), jnp.float32),
        scratch_shapes=[pltpu.VMEM((B, H), jnp.float32),
                        pltpu.VMEM((T, B, 3 * H), jnp.float32)],
    )(xs, w_ih.T, w_hh.T, b_ih.reshape(1, 3 * H), b_hh.reshape(1, 3 * H))


def _deg_body(dst_hbm, ew_hbm, deg_out, dbuf, ebuf, zbuf, deg_sh):
    c = lax.axis_index("c")
    s = lax.axis_index("s")
    wid = c * NS + s
    pltpu.sync_copy(dst_hbm.at[pl.ds(wid * CPT, CPT)], dbuf)
    pltpu.sync_copy(ew_hbm.at[pl.ds(wid * CPT, CPT)], ebuf)

    def zrow(i, carry):
        zbuf[pl.ds(i * 16, 16)] = jnp.zeros((16,), jnp.float32)
        return carry

    lax.fori_loop(0, SLICE // 16, zrow, 0)
    pltpu.sync_copy(zbuf, deg_sh.at[pl.ds(s * SLICE, SLICE)])
    plsc.subcore_barrier()

    def chunk(j, carry):
        pltpu.sync_copy(ebuf.at[j], deg_sh.at[dbuf.at[j]], add=True)
        return carry

    lax.fori_loop(0, CPT, chunk, 0)
    plsc.subcore_barrier()
    pltpu.sync_copy(deg_sh.at[pl.ds(s * SLICE, SLICE)],
                    deg_out.at[c, pl.ds(s * SLICE, SLICE)])


_deg_kernel = pl.kernel(
    _deg_body,
    out_type=jax.ShapeDtypeStruct((NC, NPAD), jnp.float32),
    mesh=_MESH,
    scratch_types=[
        pltpu.VMEM((CPT, CHUNK), jnp.int32),
        pltpu.VMEM((CPT, CHUNK), jnp.float32),
        pltpu.VMEM((SLICE,), jnp.float32),
        pltpu.VMEM_SHARED((NPAD,), jnp.float32),
    ],
)


def _scat_body(src_hbm, dst_hbm, ew_hbm, dinv_hbm, y_hbm, s_out,
               sbuf, dbuf, ebuf, rows0, rows1, dv0, dv1, s_sh,
               sem0, sem1, sd0, sd1):
    c = lax.axis_index("c")
    s = lax.axis_index("s")
    wid = c * NS + s

    def zrow(i, carry):
        for q in range(8):
            rows0[i, pl.ds(q * 16, 16)] = jnp.zeros((16,), jnp.float32)
        return carry

    lax.fori_loop(0, CHUNK, zrow, 0)

    def zcopy(q, carry):
        pltpu.sync_copy(rows0, s_sh.at[pl.ds(s * SLICE + q * CHUNK, CHUNK)])
        return carry

    lax.fori_loop(0, SLICE // CHUNK, zcopy, 0)

    plsc.subcore_barrier()

    nstage = CPT // SPC
    for p in range(nstage):
        base = wid * CPT + p * SPC
        pltpu.sync_copy(src_hbm.at[pl.ds(base, SPC)], sbuf)
        pltpu.sync_copy(dst_hbm.at[pl.ds(base, SPC)], dbuf)
        pltpu.sync_copy(ew_hbm.at[pl.ds(base, SPC)], ebuf)

        pltpu.make_async_copy(y_hbm.at[sbuf.at[0]], rows0, sem0).start()
        pltpu.make_async_copy(dinv_hbm.at[sbuf.at[0]], dv0, sd0).start()

        def process(j, rows_b, sem_b, dv_b, sd_b, rows_n, sem_n, dv_n, sd_n):
            @pl.when(j < SPC - 1)
            def _():
                pltpu.make_async_copy(y_hbm.at[sbuf.at[j + 1]], rows_n,
                                      sem_n).start()
                pltpu.make_async_copy(dinv_hbm.at[sbuf.at[j + 1]], dv_n,
                                      sd_n).start()

            pltpu.make_async_copy(y_hbm.at[sbuf.at[j]], rows_b, sem_b).wait()
            pltpu.make_async_copy(dinv_hbm.at[sbuf.at[j]], dv_b, sd_b).wait()

            def scale(k, carry2):
                cv = ebuf[j, pl.ds(k * 16, 16)] * dv_b[pl.ds(k * 16, 16)]
                for l in range(16):
                    coef = cv[l]
                    i = k * 16 + l
                    for q in range(8):
                        rows_b[i, pl.ds(q * 16, 16)] = (
                            rows_b[i, pl.ds(q * 16, 16)] * coef)
                return carry2

            lax.fori_loop(0, CHUNK // 16, scale, 0)
            pltpu.sync_copy(rows_b, s_sh.at[dbuf.at[j]], add=True)

        def pair(jj, carry):
            j = jj * 2
            process(j, rows0, sem0, dv0, sd0, rows1, sem1, dv1, sd1)
            process(j + 1, rows1, sem1, dv1, sd1, rows0, sem0, dv0, sd0)
            return carry

        lax.fori_loop(0, SPC // 2, pair, 0)

    plsc.subcore_barrier()
    pltpu.sync_copy(s_sh.at[pl.ds(s * SLICE, SLICE)],
                    s_out.at[c, pl.ds(s * SLICE, SLICE)])


_scat_kernel = pl.kernel(
    _scat_body,
    out_type=jax.ShapeDtypeStruct((NC, NPAD, H), jnp.float32),
    mesh=_MESH,
    scratch_types=[
        pltpu.VMEM((SPC, CHUNK), jnp.int32),
        pltpu.VMEM((SPC, CHUNK), jnp.int32),
        pltpu.VMEM((SPC, CHUNK), jnp.float32),
        pltpu.VMEM((CHUNK, H), jnp.float32),
        pltpu.VMEM((CHUNK, H), jnp.float32),
        pltpu.VMEM((CHUNK,), jnp.float32),
        pltpu.VMEM((CHUNK,), jnp.float32),
        pltpu.VMEM_SHARED((NPAD, H), jnp.float32),
        pltpu.SemaphoreType.DMA,
        pltpu.SemaphoreType.DMA,
        pltpu.SemaphoreType.DMA,
        pltpu.SemaphoreType.DMA,
    ],
)


def _dinv_body(deg_ref, out_ref):
    d = deg_ref[0] + deg_ref[1] + 1.0
    out_ref[...] = lax.rsqrt(jnp.maximum(d, 1e-12))


def _dinv(deg):
    out = pl.pallas_call(
        _dinv_body,
        out_shape=jax.ShapeDtypeStruct((NPAD // 128, 128), jnp.float32),
    )(deg.reshape(NC, NPAD // 128, 128))
    return out.reshape(NPAD)


def _final_body(s_ref, dinv_ref, h_ref, wg_ref, out_ref):
    dinv = dinv_ref[:N, :]
    z = dinv * (s_ref[0, :N, :] + s_ref[1, :N, :] + dinv * h_ref[...])
    zw = jnp.dot(z, wg_ref[...], preferred_element_type=jnp.float32)
    out_ref[...] = jnp.swapaxes(zw.reshape(T, B, H), 0, 1)


def _final(s_parts, dinv, hmat, w_gcn):
    return pl.pallas_call(
        _final_body,
        out_shape=jax.ShapeDtypeStruct((B, T, H), jnp.float32),
    )(s_parts, dinv.reshape(NPAD, 1), hmat, w_gcn.T)


def kernel(gru_input, edge_index_batch, edge_attr_batch, batch, w_ih, w_hh,
           b_ih, b_hh, w_gcn):
    xs = jnp.swapaxes(gru_input, 0, 1)
    h_tb = _gru(xs, w_ih, w_hh, b_ih, b_hh)
    hmat = h_tb.reshape(N, H)

    pad = E_PAD - edge_index_batch.shape[1]
    pidx = jnp.arange(pad, dtype=jnp.int32)
    s0 = edge_index_batch[0].astype(jnp.int32)
    d0 = edge_index_batch[1].astype(jnp.int32)
    src = jnp.concatenate(
        [(s0 % T) * B + s0 // T, pidx % N]).reshape(NTILES * CPT, CHUNK)
    dst = jnp.concatenate(
        [(d0 % T) * B + d0 // T, pidx % NPAD]).reshape(NTILES * CPT, CHUNK)
    ew = jnp.pad(edge_attr_batch, (0, pad)).reshape(NTILES * CPT, CHUNK)

    deg = _deg_kernel(dst, ew)
    dinv = _dinv(deg)
    s_parts = _scat_kernel(src, dst, ew, dinv, hmat)
    return _final(s_parts, dinv, hmat, w_gcn)


# speedup vs baseline: 1.1155x; 1.1155x over previous
import functools

import jax
import jax.numpy as jnp
from jax import lax
from jax.experimental import pallas as pl
from jax.experimental.pallas import tpu as pltpu
from jax.experimental.pallas import tpu_sc as plsc

B = 100
T = 100
D = 128
H = 128
N = B * T

NC = 2
NS = 16
NTILES = NC * NS
CHUNK = 128
CPT = 80
SPC = 40
EPT = CPT * CHUNK
E_PAD = NTILES * EPT
NPAD = 10240
SLICE = NPAD // NS

_MESH = plsc.VectorSubcoreMesh(core_axis_name="c", subcore_axis_name="s")


def _gru_body(xs_ref, wih_ref, whh_ref, bih_ref, bhh_ref,
              out_ref, h_ref, gi_ref):
    gi_ref[...] = (jnp.dot(
        xs_ref[...].reshape(T * B, D), wih_ref[...],
        preferred_element_type=jnp.float32) + bih_ref[...]).reshape(
            T, B, 3 * H)
    h_ref[...] = jnp.zeros((B, H), dtype=jnp.float32)

    def step(t, carry):
        gi = gi_ref[t]
        gh = jnp.dot(h_ref[...], whh_ref[...], preferred_element_type=jnp.float32) + bhh_ref[...]
        r = jax.nn.sigmoid(gi[:, 0:H] + gh[:, 0:H])
        z = jax.nn.sigmoid(gi[:, H:2 * H] + gh[:, H:2 * H])
        n = jnp.tanh(gi[:, 2 * H:3 * H] + r * gh[:, 2 * H:3 * H])
        h_new = (1.0 - z) * n + z * h_ref[...]
        h_ref[...] = h_new
        out_ref[t] = h_new
        return carry

    lax.fori_loop(0, T, step, 0)


def _gru(xs, w_ih, w_hh, b_ih, b_hh):
    return pl.pallas_call(
        _gru_body,
        out_shape=jax.ShapeDtypeStruct((T, B, H), jnp.float32),
        scratch_shapes=[pltpu.VMEM((B, H), jnp.float32),
                        pltpu.VMEM((T, B, 3 * H), jnp.float32)],
    )(xs, w_ih.T, w_hh.T, b_ih.reshape(1, 3 * H), b_hh.reshape(1, 3 * H))


def _deg_body(dst_hbm, ew_hbm, deg_out, dbuf, ebuf, zbuf, deg_sh):
    c = lax.axis_index("c")
    s = lax.axis_index("s")
    wid = c * NS + s
    pltpu.sync_copy(dst_hbm.at[pl.ds(wid * CPT, CPT)], dbuf)
    pltpu.sync_copy(ew_hbm.at[pl.ds(wid * CPT, CPT)], ebuf)

    def zrow(i, carry):
        zbuf[pl.ds(i * 16, 16)] = jnp.zeros((16,), jnp.float32)
        return carry

    lax.fori_loop(0, SLICE // 16, zrow, 0)
    pltpu.sync_copy(zbuf, deg_sh.at[pl.ds(s * SLICE, SLICE)])
    plsc.subcore_barrier()

    def chunk(j, carry):
        pltpu.sync_copy(ebuf.at[j], deg_sh.at[dbuf.at[j]], add=True)
        return carry

    lax.fori_loop(0, CPT, chunk, 0)
    plsc.subcore_barrier()
    pltpu.sync_copy(deg_sh.at[pl.ds(s * SLICE, SLICE)],
                    deg_out.at[c, pl.ds(s * SLICE, SLICE)])


_deg_kernel = pl.kernel(
    _deg_body,
    out_type=jax.ShapeDtypeStruct((NC, NPAD), jnp.float32),
    mesh=_MESH,
    scratch_types=[
        pltpu.VMEM((CPT, CHUNK), jnp.int32),
        pltpu.VMEM((CPT, CHUNK), jnp.float32),
        pltpu.VMEM((SLICE,), jnp.float32),
        pltpu.VMEM_SHARED((NPAD,), jnp.float32),
    ],
)


def _scat_body(src_hbm, dst_hbm, ew_hbm, dinv_hbm, y_hbm, s_out,
               sbuf, dbuf, ebuf, rows0, rows1, dv0, dv1, s_sh,
               sem0, sem1, sd0, sd1):
    c = lax.axis_index("c")
    s = lax.axis_index("s")
    wid = c * NS + s

    def zrow(i, carry):
        for q in range(8):
            rows0[i, pl.ds(q * 16, 16)] = jnp.zeros((16,), jnp.float32)
        return carry

    lax.fori_loop(0, CHUNK, zrow, 0)

    def zcopy(q, carry):
        pltpu.sync_copy(rows0, s_sh.at[pl.ds(s * SLICE + q * CHUNK, CHUNK)])
        return carry

    lax.fori_loop(0, SLICE // CHUNK, zcopy, 0)

    plsc.subcore_barrier()

    nstage = CPT // SPC
    for p in range(nstage):
        base = wid * CPT + p * SPC
        pltpu.sync_copy(src_hbm.at[pl.ds(base, SPC)], sbuf)
        pltpu.sync_copy(dst_hbm.at[pl.ds(base, SPC)], dbuf)
        pltpu.sync_copy(ew_hbm.at[pl.ds(base, SPC)], ebuf)

        pltpu.make_async_copy(y_hbm.at[sbuf.at[0]], rows0, sem0).start()
        pltpu.make_async_copy(dinv_hbm.at[sbuf.at[0]], dv0, sd0).start()

        def process(j, rows_b, sem_b, dv_b, sd_b, rows_n, sem_n, dv_n, sd_n):
            @pl.when(j < SPC - 1)
            def _():
                pltpu.make_async_copy(y_hbm.at[sbuf.at[j + 1]], rows_n,
                                      sem_n).start()
                pltpu.make_async_copy(dinv_hbm.at[sbuf.at[j + 1]], dv_n,
                                      sd_n).start()

            pltpu.make_async_copy(y_hbm.at[sbuf.at[j]], rows_b, sem_b).wait()
            pltpu.make_async_copy(dinv_hbm.at[sbuf.at[j]], dv_b, sd_b).wait()

            def scale(k, carry2):
                cv = ebuf[j, pl.ds(k * 16, 16)] * dv_b[pl.ds(k * 16, 16)]
                for l in range(16):
                    coef = cv[l]
                    i = k * 16 + l
                    for q in range(8):
                        rows_b[i, pl.ds(q * 16, 16)] = (
                            rows_b[i, pl.ds(q * 16, 16)] * coef)
                return carry2

            lax.fori_loop(0, CHUNK // 16, scale, 0)
            pltpu.sync_copy(rows_b, s_sh.at[dbuf.at[j]], add=True)

        def pair(jj, carry):
            j = jj * 2
            process(j, rows0, sem0, dv0, sd0, rows1, sem1, dv1, sd1)
            process(j + 1, rows1, sem1, dv1, sd1, rows0, sem0, dv0, sd0)
            return carry

        lax.fori_loop(0, SPC // 2, pair, 0)

    plsc.subcore_barrier()
    pltpu.sync_copy(s_sh.at[pl.ds(s * SLICE, SLICE)],
                    s_out.at[c, pl.ds(s * SLICE, SLICE)])


_scat_kernel = pl.kernel(
    _scat_body,
    out_type=jax.ShapeDtypeStruct((NC, NPAD, H), jnp.float32),
    mesh=_MESH,
    scratch_types=[
        pltpu.VMEM((SPC, CHUNK), jnp.int32),
        pltpu.VMEM((SPC, CHUNK), jnp.int32),
        pltpu.VMEM((SPC, CHUNK), jnp.float32),
        pltpu.VMEM((CHUNK, H), jnp.float32),
        pltpu.VMEM((CHUNK, H), jnp.float32),
        pltpu.VMEM((CHUNK,), jnp.float32),
        pltpu.VMEM((CHUNK,), jnp.float32),
        pltpu.VMEM_SHARED((NPAD, H), jnp.float32),
        pltpu.SemaphoreType.DMA,
        pltpu.SemaphoreType.DMA,
        pltpu.SemaphoreType.DMA,
        pltpu.SemaphoreType.DMA,
    ],
)


def _dinv_body(deg_ref, out_ref):
    d = deg_ref[0] + deg_ref[1] + 1.0
    out_ref[...] = lax.rsqrt(jnp.maximum(d, 1e-12))


def _dinv(deg):
    out = pl.pallas_call(
        _dinv_body,
        out_shape=jax.ShapeDtypeStruct((NPAD // 128, 128), jnp.float32),
    )(deg.reshape(NC, NPAD // 128, 128))
    return out.reshape(NPAD)


def _final_body(s_ref, dinv_ref, h_ref, wg_ref, out_ref):
    dinv = dinv_ref[:N, :]
    z = dinv * (s_ref[0, :N, :] + s_ref[1, :N, :] + dinv * h_ref[...])
    zw = jnp.dot(z, wg_ref[...], preferred_element_type=jnp.float32)
    out_ref[...] = jnp.swapaxes(zw.reshape(T, B, H), 0, 1)


def _final(s_parts, dinv, hmat, w_gcn):
    return pl.pallas_call(
        _final_body,
        out_shape=jax.ShapeDtypeStruct((B, T, H), jnp.float32),
    )(s_parts, dinv.reshape(NPAD, 1), hmat, w_gcn.T)


def kernel(gru_input, edge_index_batch, edge_attr_batch, batch, w_ih, w_hh,
           b_ih, b_hh, w_gcn):
    xs = jnp.swapaxes(gru_input, 0, 1)
    h_tb = _gru(xs, w_ih, w_hh, b_ih, b_hh)
    hmat = h_tb.reshape(N, H)

    pad = E_PAD - edge_index_batch.shape[1]
    pidx = jnp.arange(pad, dtype=jnp.int32)
    def _remap(n):
        q = (n * 5243) >> 19
        return (n - q * T) * B + q

    s0 = edge_index_batch[0].astype(jnp.int32)
    d0 = edge_index_batch[1].astype(jnp.int32)
    src = jnp.concatenate(
        [_remap(s0), pidx % N]).reshape(NTILES * CPT, CHUNK)
    dst = jnp.concatenate(
        [_remap(d0), pidx % NPAD]).reshape(NTILES * CPT, CHUNK)
    ew = jnp.pad(edge_attr_batch, (0, pad)).reshape(NTILES * CPT, CHUNK)

    deg = _deg_kernel(dst, ew)
    dinv = _dinv(deg)
    s_parts = _scat_kernel(src, dst, ew, dinv, hmat)
    return _final(s_parts, dinv, hmat, w_gcn)


# speedup vs baseline: 1.1452x; 1.0266x over previous
import functools

import jax
import jax.numpy as jnp
from jax import lax
from jax.experimental import pallas as pl
from jax.experimental.pallas import tpu as pltpu
from jax.experimental.pallas import tpu_sc as plsc

B = 100
T = 100
D = 128
H = 128
N = B * T

NC = 2
NS = 16
NTILES = NC * NS
CHUNK = 128
CPT = 80
SPC = 40
EPT = CPT * CHUNK
E_PAD = NTILES * EPT
NPAD = 10240
SLICE = NPAD // NS

_MESH = plsc.VectorSubcoreMesh(core_axis_name="c", subcore_axis_name="s")


def _gru_body(xs_ref, wih_ref, whh_ref, bih_ref, bhh_ref,
              out_ref, h_ref, gi_ref):
    gi_ref[...] = (jnp.dot(
        xs_ref[...].reshape(T * B, D), wih_ref[...],
        preferred_element_type=jnp.float32) + bih_ref[...]).reshape(
            T, B, 3 * H)
    h_ref[...] = jnp.zeros((B, H), dtype=jnp.float32)

    def step(t, carry):
        gi = gi_ref[t]
        gh = jnp.dot(h_ref[...], whh_ref[...], preferred_element_type=jnp.float32) + bhh_ref[...]
        r = jax.nn.sigmoid(gi[:, 0:H] + gh[:, 0:H])
        z = jax.nn.sigmoid(gi[:, H:2 * H] + gh[:, H:2 * H])
        n = jnp.tanh(gi[:, 2 * H:3 * H] + r * gh[:, 2 * H:3 * H])
        h_new = (1.0 - z) * n + z * h_ref[...]
        h_ref[...] = h_new
        out_ref[pl.ds(t * B, B), :] = h_new
        return carry

    lax.fori_loop(0, T, step, 0)


def _gru(xs, w_ih, w_hh, b_ih, b_hh):
    return pl.pallas_call(
        _gru_body,
        out_shape=jax.ShapeDtypeStruct((N, H), jnp.float32),
        scratch_shapes=[pltpu.VMEM((B, H), jnp.float32),
                        pltpu.VMEM((T, B, 3 * H), jnp.float32)],
    )(xs, w_ih.T, w_hh.T, b_ih.reshape(1, 3 * H), b_hh.reshape(1, 3 * H))


def _deg_body(dst_hbm, ew_hbm, deg_out, dbuf, ebuf, zbuf, deg_sh):
    c = lax.axis_index("c")
    s = lax.axis_index("s")
    wid = c * NS + s
    pltpu.sync_copy(dst_hbm.at[pl.ds(wid * CPT, CPT)], dbuf)
    pltpu.sync_copy(ew_hbm.at[pl.ds(wid * CPT, CPT)], ebuf)

    def zrow(i, carry):
        zbuf[pl.ds(i * 16, 16)] = jnp.zeros((16,), jnp.float32)
        return carry

    lax.fori_loop(0, SLICE // 16, zrow, 0)
    pltpu.sync_copy(zbuf, deg_sh.at[pl.ds(s * SLICE, SLICE)])
    plsc.subcore_barrier()

    def chunk(j, carry):
        pltpu.sync_copy(ebuf.at[j], deg_sh.at[dbuf.at[j]], add=True)
        return carry

    lax.fori_loop(0, CPT, chunk, 0)
    plsc.subcore_barrier()
    pltpu.sync_copy(deg_sh.at[pl.ds(s * SLICE, SLICE)],
                    deg_out.at[c, pl.ds(s * SLICE, SLICE)])


_deg_kernel = pl.kernel(
    _deg_body,
    out_type=jax.ShapeDtypeStruct((NC, NPAD), jnp.float32),
    mesh=_MESH,
    scratch_types=[
        pltpu.VMEM((CPT, CHUNK), jnp.int32),
        pltpu.VMEM((CPT, CHUNK), jnp.float32),
        pltpu.VMEM((SLICE,), jnp.float32),
        pltpu.VMEM_SHARED((NPAD,), jnp.float32),
    ],
)


def _scat_body(src_hbm, dst_hbm, ew_hbm, dinv_hbm, y_hbm, s_out,
               sbuf, dbuf, ebuf, rows0, rows1, dv0, dv1, s_sh,
               sem0, sem1, sd0, sd1):
    c = lax.axis_index("c")
    s = lax.axis_index("s")
    wid = c * NS + s

    def zrow(i, carry):
        for q in range(8):
            rows0[i, pl.ds(q * 16, 16)] = jnp.zeros((16,), jnp.float32)
        return carry

    lax.fori_loop(0, CHUNK, zrow, 0)

    def zcopy(q, carry):
        pltpu.sync_copy(rows0, s_sh.at[pl.ds(s * SLICE + q * CHUNK, CHUNK)])
        return carry

    lax.fori_loop(0, SLICE // CHUNK, zcopy, 0)

    plsc.subcore_barrier()

    nstage = CPT // SPC
    for p in range(nstage):
        base = wid * CPT + p * SPC
        pltpu.sync_copy(src_hbm.at[pl.ds(base, SPC)], sbuf)
        pltpu.sync_copy(dst_hbm.at[pl.ds(base, SPC)], dbuf)
        pltpu.sync_copy(ew_hbm.at[pl.ds(base, SPC)], ebuf)

        pltpu.make_async_copy(y_hbm.at[sbuf.at[0]], rows0, sem0).start()
        pltpu.make_async_copy(dinv_hbm.at[sbuf.at[0]], dv0, sd0).start()

        def process(j, rows_b, sem_b, dv_b, sd_b, rows_n, sem_n, dv_n, sd_n):
            @pl.when(j < SPC - 1)
            def _():
                pltpu.make_async_copy(y_hbm.at[sbuf.at[j + 1]], rows_n,
                                      sem_n).start()
                pltpu.make_async_copy(dinv_hbm.at[sbuf.at[j + 1]], dv_n,
                                      sd_n).start()

            pltpu.make_async_copy(y_hbm.at[sbuf.at[j]], rows_b, sem_b).wait()
            pltpu.make_async_copy(dinv_hbm.at[sbuf.at[j]], dv_b, sd_b).wait()

            def scale(k, carry2):
                cv = ebuf[j, pl.ds(k * 16, 16)] * dv_b[pl.ds(k * 16, 16)]
                for l in range(16):
                    coef = cv[l]
                    i = k * 16 + l
                    for q in range(8):
                        rows_b[i, pl.ds(q * 16, 16)] = (
                            rows_b[i, pl.ds(q * 16, 16)] * coef)
                return carry2

            lax.fori_loop(0, CHUNK // 16, scale, 0)
            pltpu.sync_copy(rows_b, s_sh.at[dbuf.at[j]], add=True)

        def pair(jj, carry):
            j = jj * 2
            process(j, rows0, sem0, dv0, sd0, rows1, sem1, dv1, sd1)
            process(j + 1, rows1, sem1, dv1, sd1, rows0, sem0, dv0, sd0)
            return carry

        lax.fori_loop(0, SPC // 2, pair, 0)

    plsc.subcore_barrier()
    pltpu.sync_copy(s_sh.at[pl.ds(s * SLICE, SLICE)],
                    s_out.at[c, pl.ds(s * SLICE, SLICE)])


_scat_kernel = pl.kernel(
    _scat_body,
    out_type=jax.ShapeDtypeStruct((NC, NPAD, H), jnp.float32),
    mesh=_MESH,
    scratch_types=[
        pltpu.VMEM((SPC, CHUNK), jnp.int32),
        pltpu.VMEM((SPC, CHUNK), jnp.int32),
        pltpu.VMEM((SPC, CHUNK), jnp.float32),
        pltpu.VMEM((CHUNK, H), jnp.float32),
        pltpu.VMEM((CHUNK, H), jnp.float32),
        pltpu.VMEM((CHUNK,), jnp.float32),
        pltpu.VMEM((CHUNK,), jnp.float32),
        pltpu.VMEM_SHARED((NPAD, H), jnp.float32),
        pltpu.SemaphoreType.DMA,
        pltpu.SemaphoreType.DMA,
        pltpu.SemaphoreType.DMA,
        pltpu.SemaphoreType.DMA,
    ],
)


def _dinv_body(deg_ref, out_ref):
    d = deg_ref[0] + deg_ref[1] + 1.0
    out_ref[...] = lax.rsqrt(jnp.maximum(d, 1e-12))


def _dinv(deg):
    out = pl.pallas_call(
        _dinv_body,
        out_shape=jax.ShapeDtypeStruct((NPAD // 128, 128), jnp.float32),
    )(deg.reshape(NC, NPAD // 128, 128))
    return out.reshape(NPAD)


def _final_body(s_ref, dinv_ref, h_ref, wg_ref, out_ref):
    dinv = dinv_ref[:N, :]
    z = dinv * (s_ref[0, :N, :] + s_ref[1, :N, :] + dinv * h_ref[...])
    zw = jnp.dot(z, wg_ref[...], preferred_element_type=jnp.float32)
    out_ref[...] = jnp.swapaxes(zw.reshape(T, B, H), 0, 1)


def _final(s_parts, dinv, hmat, w_gcn):
    return pl.pallas_call(
        _final_body,
        out_shape=jax.ShapeDtypeStruct((B, T, H), jnp.float32),
    )(s_parts, dinv.reshape(NPAD, 1), hmat, w_gcn.T)


def kernel(gru_input, edge_index_batch, edge_attr_batch, batch, w_ih, w_hh,
           b_ih, b_hh, w_gcn):
    xs = jnp.swapaxes(gru_input, 0, 1)
    hmat = _gru(xs, w_ih, w_hh, b_ih, b_hh)

    pad = E_PAD - edge_index_batch.shape[1]
    pidx = jnp.arange(pad, dtype=jnp.int32)
    def _remap(n):
        q = (n * 5243) >> 19
        return (n - q * T) * B + q

    s0 = edge_index_batch[0].astype(jnp.int32)
    d0 = edge_index_batch[1].astype(jnp.int32)
    src = jnp.concatenate(
        [_remap(s0), pidx % N]).reshape(NTILES * CPT, CHUNK)
    dst = jnp.concatenate(
        [_remap(d0), pidx % NPAD]).reshape(NTILES * CPT, CHUNK)
    ew = jnp.pad(edge_attr_batch, (0, pad)).reshape(NTILES * CPT, CHUNK)

    deg = _deg_kernel(dst, ew)
    dinv = _dinv(deg)
    s_parts = _scat_kernel(src, dst, ew, dinv, hmat)
    return _final(s_parts, dinv, hmat, w_gcn)
